# baseline (speedup 1.0000x reference)
import jax
import jax.numpy as jnp
from jax.experimental import pallas as pl
from jax.experimental.pallas import tpu as pltpu

_PATCHES_PER_BATCH = 16
_K = 16
_W_NORMAL = 1.0
_W_SURFVAR = 1.0
_NEWTON_ITERS = 20
_PAIR = 4
_HI = jax.lax.Precision.HIGHEST


def _one_patch(rx, ry, rz, cx, cy, cz):
    dx = cx - rx
    dy = cy - ry
    dz = cz - rz
    d2 = dx * dx + dy * dy + dz * dz

    work = jnp.where(d2 == 0.0, jnp.inf, d2)
    for _ in range(_K - 1):
        m = jnp.min(work, axis=0, keepdims=True)
        work = jnp.where(work == m, jnp.inf, work)
    wf = jnp.where(work == d2, 0.0, 1.0)

    ones = jnp.ones_like(rx)
    ft = jnp.concatenate(
        [rx, ry, rz, rx * rx, ry * ry, rz * rz,
         rx * ry, rx * rz, ry * rz, ones], axis=0)
    R = jax.lax.dot_general(ft, wf, (((1,), (0,)), ((), ())),
                            precision=_HI,
                            preferred_element_type=jnp.float32)
    sxr = R[0:1, :]
    syr = R[1:2, :]
    szr = R[2:3, :]
    qxx = R[3:4, :]
    qyy = R[4:5, :]
    qzz = R[5:6, :]
    qxy = R[6:7, :]
    qxz = R[7:8, :]
    qyz = R[8:9, :]
    cnt = R[9:10, :]

    a11 = qxx - rx * (2.0 * sxr - cnt * rx)
    a22 = qyy - ry * (2.0 * syr - cnt * ry)
    a33 = qzz - rz * (2.0 * szr - cnt * rz)
    a12 = qxy - rx * syr - ry * sxr + cnt * rx * ry
    a13 = qxz - rx * szr - rz * sxr + cnt * rx * rz
    a23 = qyz - ry * szr - rz * syr + cnt * ry * rz

    tr = a11 + a22 + a33
    inv = 1.0 / jnp.maximum(tr, 1e-30)
    b11 = a11 * inv
    b12 = a12 * inv
    b13 = a13 * inv
    b22 = a22 * inv
    b23 = a23 * inv
    b33 = a33 * inv

    ctr = b11 + b22 + b33
    c1 = (b11 * b22 - b12 * b12) + (b11 * b33 - b13 * b13) + (b22 * b33 - b23 * b23)
    c0 = (b11 * (b22 * b33 - b23 * b23)
          - b12 * (b12 * b33 - b23 * b13)
          + b13 * (b12 * b23 - b22 * b13))

    lam = jnp.zeros_like(c0)
    for _ in range(_NEWTON_ITERS):
        f = ((ctr - lam) * lam - c1) * lam + c0
        fp = (2.0 * ctr - 3.0 * lam) * lam - c1
        lam = lam - f / jnp.minimum(fp, -1e-30)
        lam = jnp.clip(lam, 0.0, 0.33334)
    sv = lam

    alpha = ctr - lam
    beta = c1 - lam * alpha
    s11 = b11 * b11 + b12 * b12 + b13 * b13
    s12 = b11 * b12 + b12 * b22 + b13 * b23
    s13 = b11 * b13 + b12 * b23 + b13 * b33
    s22 = b12 * b12 + b22 * b22 + b23 * b23
    s23 = b12 * b13 + b22 * b23 + b23 * b33
    s33 = b13 * b13 + b23 * b23 + b33 * b33
    m11 = s11 - alpha * b11 + beta
    m12 = s12 - alpha * b12
    m13 = s13 - alpha * b13
    m22 = s22 - alpha * b22 + beta
    m23 = s23 - alpha * b23
    m33 = s33 - alpha * b33 + beta

    n1 = m11 * m11 + m12 * m12 + m13 * m13
    n2 = m12 * m12 + m22 * m22 + m23 * m23
    n3 = m13 * m13 + m23 * m23 + m33 * m33
    use1 = (n1 >= n2) & (n1 >= n3)
    use2 = jnp.logical_not(use1) & (n2 >= n3)
    vx = jnp.where(use1, m11, jnp.where(use2, m12, m13))
    vy = jnp.where(use1, m12, jnp.where(use2, m22, m23))
    vz = jnp.where(use1, m13, jnp.where(use2, m23, m33))
    nn = vx * vx + vy * vy + vz * vz
    invn = jax.lax.rsqrt(jnp.maximum(nn, 1e-38))
    nx = jnp.abs(vx) * invn
    ny = jnp.abs(vy) * invn
    nz = jnp.abs(vz) * invn

    mm = jnp.float32(rx.shape[1])
    ss = (jnp.sum(nx * nx) - jnp.sum(nx) ** 2 / mm
          + jnp.sum(ny * ny) - jnp.sum(ny) ** 2 / mm
          + jnp.sum(nz * nz) - jnp.sum(nz) ** 2 / mm)
    svsum = jnp.sum(sv)
    return ss, svsum


def _patch_body(xT_ref, xC_ref, out_ref):
    xt = xT_ref[...]
    xc = xC_ref[...]
    lane = jax.lax.broadcasted_iota(jnp.int32, (1, 1, 128), 2)

    @pl.when(pl.program_id(0) == 0)
    def _init():
        out_ref[...] = jnp.zeros((1, 1, 128), jnp.float32)

    acc = jnp.zeros((1, 1, 128), jnp.float32)
    for s in range(_PAIR):
        rx = xt[s, 0:1, :]
        ry = xt[s, 1:2, :]
        rz = xt[s, 2:3, :]
        cx = xc[s, 0]
        cy = xc[s, 1]
        cz = xc[s, 2]
        ss, svsum = _one_patch(rx, ry, rz, cx, cy, cz)
        acc = acc + jnp.where(lane == 0, ss,
                              jnp.where(lane == 1, svsum, 0.0))
    out_ref[...] += acc


def kernel(pointCloud):
    B, N, _ = pointCloud.shape
    P = B * _PATCHES_PER_BATCH
    M = N // _PATCHES_PER_BATCH
    x = pointCloud.reshape(P, M, 3)
    xT = jnp.transpose(x, (0, 2, 1))
    xC = xT[..., None]

    partials = pl.pallas_call(
        _patch_body,
        grid=(P // _PAIR,),
        in_specs=[
            pl.BlockSpec((_PAIR, 3, M), lambda p: (p, 0, 0)),
            pl.BlockSpec((_PAIR, 3, M, 1), lambda p: (p, 0, 0, 0)),
        ],
        out_specs=pl.BlockSpec((1, 1, 128), lambda p: (0, 0, 0)),
        out_shape=jax.ShapeDtypeStruct((1, 1, 128), jnp.float32),
        compiler_params=pltpu.CompilerParams(
            dimension_semantics=("arbitrary",),
        ),
    )(xT, xC)

    loss = (partials[0, 0, 0] / (P * M * 3) * _W_NORMAL
            + partials[0, 0, 1] / (P * M) * _W_SURFVAR)
    return loss.astype(jnp.float32)


# speedup vs baseline: 1.0389x; 1.0389x over previous
import jax
import jax.numpy as jnp
from jax.experimental import pallas as pl
from jax.experimental.pallas import tpu as pltpu

_PATCHES_PER_BATCH = 16
_K = 16
_W_NORMAL = 1.0
_W_SURFVAR = 1.0
_NEWTON_ITERS = 20
_PAIR = 8
_HI = jax.lax.Precision.HIGHEST


def _one_patch(rx, ry, rz, cx, cy, cz):
    d2 = (cx - rx) ** 2 + (cy - ry) ** 2 + (cz - rz) ** 2

    work = jnp.where(d2 == 0.0, jnp.inf, d2)
    for _ in range(_K - 1):
        m = jnp.min(work, axis=0, keepdims=True)
        work = jnp.where(work == m, jnp.inf, work)
    wf = jnp.where(work == d2, 0.0, 1.0)

    ones = jnp.ones_like(rx)
    ft = jnp.concatenate(
        [rx, ry, rz, rx * rx, ry * ry, rz * rz,
         rx * ry, rx * rz, ry * rz, ones], axis=0)
    R = jax.lax.dot_general(ft, wf, (((1,), (0,)), ((), ())),
                            precision=_HI,
                            preferred_element_type=jnp.float32)
    sxr = R[0:1, :]
    syr = R[1:2, :]
    szr = R[2:3, :]
    qxx = R[3:4, :]
    qyy = R[4:5, :]
    qzz = R[5:6, :]
    qxy = R[6:7, :]
    qxz = R[7:8, :]
    qyz = R[8:9, :]
    cnt = R[9:10, :]

    a11 = qxx - rx * (2.0 * sxr - cnt * rx)
    a22 = qyy - ry * (2.0 * syr - cnt * ry)
    a33 = qzz - rz * (2.0 * szr - cnt * rz)
    a12 = qxy - rx * syr - ry * sxr + cnt * rx * ry
    a13 = qxz - rx * szr - rz * sxr + cnt * rx * rz
    a23 = qyz - ry * szr - rz * syr + cnt * ry * rz

    tr = a11 + a22 + a33
    inv = 1.0 / jnp.maximum(tr, 1e-30)
    b11 = a11 * inv
    b12 = a12 * inv
    b13 = a13 * inv
    b22 = a22 * inv
    b23 = a23 * inv
    b33 = a33 * inv

    ctr = b11 + b22 + b33
    c1 = (b11 * b22 - b12 * b12) + (b11 * b33 - b13 * b13) + (b22 * b33 - b23 * b23)
    c0 = (b11 * (b22 * b33 - b23 * b23)
          - b12 * (b12 * b33 - b23 * b13)
          + b13 * (b12 * b23 - b22 * b13))

    lam = jnp.zeros_like(c0)
    for _ in range(_NEWTON_ITERS):
        f = ((ctr - lam) * lam - c1) * lam + c0
        fp = (2.0 * ctr - 3.0 * lam) * lam - c1
        lam = lam - f / jnp.minimum(fp, -1e-30)
        lam = jnp.clip(lam, 0.0, 0.33334)
    sv = lam

    alpha = ctr - lam
    beta = c1 - lam * alpha
    s11 = b11 * b11 + b12 * b12 + b13 * b13
    s12 = b11 * b12 + b12 * b22 + b13 * b23
    s13 = b11 * b13 + b12 * b23 + b13 * b33
    s22 = b12 * b12 + b22 * b22 + b23 * b23
    s23 = b12 * b13 + b22 * b23 + b23 * b33
    s33 = b13 * b13 + b23 * b23 + b33 * b33
    m11 = s11 - alpha * b11 + beta
    m12 = s12 - alpha * b12
    m13 = s13 - alpha * b13
    m22 = s22 - alpha * b22 + beta
    m23 = s23 - alpha * b23
    m33 = s33 - alpha * b33 + beta

    n1 = m11 * m11 + m12 * m12 + m13 * m13
    n2 = m12 * m12 + m22 * m22 + m23 * m23
    n3 = m13 * m13 + m23 * m23 + m33 * m33
    use1 = (n1 >= n2) & (n1 >= n3)
    use2 = jnp.logical_not(use1) & (n2 >= n3)
    vx = jnp.where(use1, m11, jnp.where(use2, m12, m13))
    vy = jnp.where(use1, m12, jnp.where(use2, m22, m23))
    vz = jnp.where(use1, m13, jnp.where(use2, m23, m33))
    nn = vx * vx + vy * vy + vz * vz
    invn = jax.lax.rsqrt(jnp.maximum(nn, 1e-38))
    nx = jnp.abs(vx) * invn
    ny = jnp.abs(vy) * invn
    nz = jnp.abs(vz) * invn

    mm = jnp.float32(rx.shape[1])
    ss = (jnp.sum(nx * nx) - jnp.sum(nx) ** 2 / mm
          + jnp.sum(ny * ny) - jnp.sum(ny) ** 2 / mm
          + jnp.sum(nz * nz) - jnp.sum(nz) ** 2 / mm)
    svsum = jnp.sum(sv)
    return ss, svsum


def _patch_body(xT_ref, xC_ref, out_ref):
    xt = xT_ref[...]
    xc = xC_ref[...]
    lane = jax.lax.broadcasted_iota(jnp.int32, (1, 1, 128), 2)
    for s in range(_PAIR):
        rx = xt[s, 0:1, :]
        ry = xt[s, 1:2, :]
        rz = xt[s, 2:3, :]
        cx = xc[s, 0]
        cy = xc[s, 1]
        cz = xc[s, 2]
        ss, svsum = _one_patch(rx, ry, rz, cx, cy, cz)
        out_ref[s : s + 1] = jnp.where(lane == 0, ss,
                                       jnp.where(lane == 1, svsum, 0.0))


def kernel(pointCloud):
    B, N, _ = pointCloud.shape
    P = B * _PATCHES_PER_BATCH
    M = N // _PATCHES_PER_BATCH
    x = pointCloud.reshape(P, M, 3)
    xT = jnp.transpose(x, (0, 2, 1))
    xC = xT[..., None]

    partials = pl.pallas_call(
        _patch_body,
        grid=(P // _PAIR,),
        in_specs=[
            pl.BlockSpec((_PAIR, 3, M), lambda p: (p, 0, 0)),
            pl.BlockSpec((_PAIR, 3, M, 1), lambda p: (p, 0, 0, 0)),
        ],
        out_specs=pl.BlockSpec((_PAIR, 1, 128), lambda p: (p, 0, 0)),
        out_shape=jax.ShapeDtypeStruct((P, 1, 128), jnp.float32),
        compiler_params=pltpu.CompilerParams(
            dimension_semantics=("arbitrary",),
        ),
    )(xT, xC)

    nss = jnp.sum(partials[:, 0, 0])
    svs = jnp.sum(partials[:, 0, 1])
    loss = nss / (P * M * 3) * _W_NORMAL + svs / (P * M) * _W_SURFVAR
    return loss.astype(jnp.float32)


# speedup vs baseline: 1.1234x; 1.0813x over previous
import jax
import jax.numpy as jnp
from jax.experimental import pallas as pl
from jax.experimental.pallas import tpu as pltpu

_PATCHES_PER_BATCH = 16
_K = 16
_W_NORMAL = 1.0
_W_SURFVAR = 1.0
_NEWTON_ITERS = 20
_PAIR = 8
_HI = jax.lax.Precision.HIGHEST


def _patch_moments(rx, ry, rz, cx, cy, cz):
    d2 = (cx - rx) ** 2 + (cy - ry) ** 2 + (cz - rz) ** 2

    work = jnp.where(d2 == 0.0, jnp.inf, d2)
    for _ in range(_K - 1):
        m = jnp.min(work, axis=0, keepdims=True)
        work = jnp.where(work == m, jnp.inf, work)
    wf = jnp.where(work == d2, 0.0, 1.0)

    ones = jnp.ones_like(rx)
    ft = jnp.concatenate(
        [rx, ry, rz, rx * rx, ry * ry, rz * rz,
         rx * ry, rx * rz, ry * rz, ones], axis=0)
    return jax.lax.dot_general(ft, wf, (((1,), (0,)), ((), ())),
                               precision=_HI,
                               preferred_element_type=jnp.float32)


def _patch_body(xT_ref, xC_ref, out_ref):
    xt = xT_ref[...]
    xc = xC_ref[...]

    moms = []
    for s in range(_PAIR):
        rxs = xt[s, 0:1, :]
        rys = xt[s, 1:2, :]
        rzs = xt[s, 2:3, :]
        cxs = xc[s, 0]
        cys = xc[s, 1]
        czs = xc[s, 2]
        moms.append(_patch_moments(rxs, rys, rzs, cxs, cys, czs))

    rx = xt[:, 0, :]
    ry = xt[:, 1, :]
    rz = xt[:, 2, :]

    def row(idx):
        return jnp.concatenate([mom[idx:idx + 1, :] for mom in moms], axis=0)

    sxr = row(0)
    syr = row(1)
    szr = row(2)
    qxx = row(3)
    qyy = row(4)
    qzz = row(5)
    qxy = row(6)
    qxz = row(7)
    qyz = row(8)
    cnt = row(9)

    a11 = qxx - rx * (2.0 * sxr - cnt * rx)
    a22 = qyy - ry * (2.0 * syr - cnt * ry)
    a33 = qzz - rz * (2.0 * szr - cnt * rz)
    a12 = qxy - rx * syr - ry * sxr + cnt * rx * ry
    a13 = qxz - rx * szr - rz * sxr + cnt * rx * rz
    a23 = qyz - ry * szr - rz * syr + cnt * ry * rz

    tr = a11 + a22 + a33
    inv = 1.0 / jnp.maximum(tr, 1e-30)
    b11 = a11 * inv
    b12 = a12 * inv
    b13 = a13 * inv
    b22 = a22 * inv
    b23 = a23 * inv
    b33 = a33 * inv

    ctr = b11 + b22 + b33
    c1 = (b11 * b22 - b12 * b12) + (b11 * b33 - b13 * b13) + (b22 * b33 - b23 * b23)
    c0 = (b11 * (b22 * b33 - b23 * b23)
          - b12 * (b12 * b33 - b23 * b13)
          + b13 * (b12 * b23 - b22 * b13))

    lam = jnp.zeros_like(c0)
    for _ in range(_NEWTON_ITERS):
        f = ((ctr - lam) * lam - c1) * lam + c0
        fp = (2.0 * ctr - 3.0 * lam) * lam - c1
        lam = lam - f / jnp.minimum(fp, -1e-30)
        lam = jnp.clip(lam, 0.0, 0.33334)
    sv = lam

    alpha = ctr - lam
    beta = c1 - lam * alpha
    s11 = b11 * b11 + b12 * b12 + b13 * b13
    s12 = b11 * b12 + b12 * b22 + b13 * b23
    s13 = b11 * b13 + b12 * b23 + b13 * b33
    s22 = b12 * b12 + b22 * b22 + b23 * b23
    s23 = b12 * b13 + b22 * b23 + b23 * b33
    s33 = b13 * b13 + b23 * b23 + b33 * b33
    m11 = s11 - alpha * b11 + beta
    m12 = s12 - alpha * b12
    m13 = s13 - alpha * b13
    m22 = s22 - alpha * b22 + beta
    m23 = s23 - alpha * b23
    m33 = s33 - alpha * b33 + beta

    n1 = m11 * m11 + m12 * m12 + m13 * m13
    n2 = m12 * m12 + m22 * m22 + m23 * m23
    n3 = m13 * m13 + m23 * m23 + m33 * m33
    use1 = (n1 >= n2) & (n1 >= n3)
    use2 = jnp.logical_not(use1) & (n2 >= n3)
    vx = jnp.where(use1, m11, jnp.where(use2, m12, m13))
    vy = jnp.where(use1, m12, jnp.where(use2, m22, m23))
    vz = jnp.where(use1, m13, jnp.where(use2, m23, m33))
    nn = vx * vx + vy * vy + vz * vz
    invn = jax.lax.rsqrt(jnp.maximum(nn, 1e-38))
    nx = jnp.abs(vx) * invn
    ny = jnp.abs(vy) * invn
    nz = jnp.abs(vz) * invn

    def psum(v):
        return jnp.sum(v, axis=1, keepdims=True)

    mm = jnp.float32(rx.shape[1])
    ss = (psum(nx * nx) - psum(nx) ** 2 / mm
          + psum(ny * ny) - psum(ny) ** 2 / mm
          + psum(nz * nz) - psum(nz) ** 2 / mm)
    svsum = psum(sv)

    lane = jax.lax.broadcasted_iota(jnp.int32, (1, 1, 128), 2)
    out_ref[...] = jnp.where(lane == 0, ss[:, :, None],
                             jnp.where(lane == 1, svsum[:, :, None], 0.0))


def kernel(pointCloud):
    B, N, _ = pointCloud.shape
    P = B * _PATCHES_PER_BATCH
    M = N // _PATCHES_PER_BATCH
    x = pointCloud.reshape(P, M, 3)
    xT = jnp.transpose(x, (0, 2, 1))
    xC = xT[..., None]

    partials = pl.pallas_call(
        _patch_body,
        grid=(P // _PAIR,),
        in_specs=[
            pl.BlockSpec((_PAIR, 3, M), lambda p: (p, 0, 0)),
            pl.BlockSpec((_PAIR, 3, M, 1), lambda p: (p, 0, 0, 0)),
        ],
        out_specs=pl.BlockSpec((_PAIR, 1, 128), lambda p: (p, 0, 0)),
        out_shape=jax.ShapeDtypeStruct((P, 1, 128), jnp.float32),
        compiler_params=pltpu.CompilerParams(
            dimension_semantics=("arbitrary",),
        ),
    )(xT, xC)

    nss = jnp.sum(partials[:, 0, 0])
    svs = jnp.sum(partials[:, 0, 1])
    loss = nss / (P * M * 3) * _W_NORMAL + svs / (P * M) * _W_SURFVAR
    return loss.astype(jnp.float32)
